# baseline (speedup 1.0000x reference)
import functools

import jax
import jax.numpy as jnp
from jax import lax
from jax.experimental import pallas as pl
from jax.experimental.pallas import tpu as pltpu
from jax.experimental.pallas import tpu_sc as plsc

N = 10000
F = 128
TM = 200
TAIL = 1600
HEAD = N - TAIL
NBH = HEAD // TM
NBT = TAIL // TM
NB = N // TM

NWORK = 32
RPW = TAIL // NWORK
NL = 16


def _sc_merge_body(a_ref, wb_ref, out_ref,
                   b0_ref, b1_ref, m_ref, wbv_ref,
                   sem_a, sem_b, sem_o):
    wid = lax.axis_index("s") * 2 + lax.axis_index("c")
    pltpu.sync_copy(wb_ref, wbv_ref)
    v0 = wbv_ref[0, :]
    v1 = wbv_ref[1, :]
    base = wid * RPW

    def in_a(k, slot):
        return pltpu.make_async_copy(
            a_ref.at[0, HEAD + base + k], b0_ref.at[slot], sem_a.at[slot])

    def in_b(k, slot):
        return pltpu.make_async_copy(
            a_ref.at[1, HEAD + base + k], b1_ref.at[slot], sem_b.at[slot])

    def out_c(k, slot):
        return pltpu.make_async_copy(
            m_ref.at[slot], out_ref.at[base + k], sem_o.at[slot])

    in_a(0, 0).start()
    in_b(0, 0).start()

    def row_step(k, _):
        slot = lax.rem(k, 2)
        nslot = lax.rem(k + 1, 2)

        @pl.when(k >= 2)
        def _():
            out_c(k - 2, slot).wait()

        in_a(k, slot).wait()
        in_b(k, slot).wait()

        @pl.when(k < RPW - 1)
        def _():
            in_a(k + 1, nslot).start()
            in_b(k + 1, nslot).start()

        def col_step(j, _):
            for c in range(25):
                sl = pl.ds((j * 25 + c) * NL, NL)
                m_ref[slot, sl] = (b0_ref[slot, sl] * v0
                                   + b1_ref[slot, sl] * v1)
            return ()

        lax.fori_loop(0, N // NL // 25, col_step, ())
        out_c(k, slot).start()
        return ()

    lax.fori_loop(0, RPW, row_step, ())
    out_c(RPW - 2, 0).wait()
    out_c(RPW - 1, 1).wait()


def _sc_merge(A, weight_b):
    wbv = jnp.broadcast_to(weight_b.reshape(2, 1), (2, NL)).astype(jnp.float32)
    mesh = plsc.VectorSubcoreMesh(core_axis_name="c", subcore_axis_name="s")
    k = functools.partial(
        pl.kernel,
        out_type=jax.ShapeDtypeStruct((TAIL, N), jnp.float32),
        mesh=mesh,
        scratch_types=[
            pltpu.VMEM((2, N), jnp.float32),
            pltpu.VMEM((2, N), jnp.float32),
            pltpu.VMEM((2, N), jnp.float32),
            pltpu.VMEM((2, NL), jnp.float32),
            pltpu.SemaphoreType.DMA((2,)),
            pltpu.SemaphoreType.DMA((2,)),
            pltpu.SemaphoreType.DMA((2,)),
        ],
    )(_sc_merge_body)
    return k(A, wbv)


def _head_body(wb_ref, a0_ref, a1_ref, f_ref, w1_ref, b1_ref, w2_ref,
               u1_ref, s2_ref, fa8_ref, s1out_ref, s1_ref):
    @pl.when(pl.program_id(0) == 0)
    def _():
        s1 = jnp.dot(f_ref[...], w1_ref[...],
                     preferred_element_type=jnp.float32).astype(jnp.bfloat16)
        s1_ref[...] = s1
        s1out_ref[...] = s1

    wb0 = wb_ref[0, 0]
    wb1 = wb_ref[1, 0]
    m = a0_ref[0] * wb0 + a1_ref[0] * wb1
    fa8_ref[...] = m.astype(jnp.float8_e4m3fn)
    mb = m.astype(jnp.bfloat16)
    u1 = jnp.dot(mb, s1_ref[...], preferred_element_type=jnp.float32)
    u1 = u1 + b1_ref[...]
    u1_ref[...] = u1
    s2_ref[...] = jnp.dot(u1.astype(jnp.bfloat16), w2_ref[...],
                          preferred_element_type=jnp.float32).astype(
                              jnp.float8_e4m3fn)


def _tail_body(mt_ref, s1_ref, b1_ref, w2_ref, u1a_ref, s2a_ref, fa8a_ref,
               u1_ref, s2_ref, fa8_ref):
    del u1a_ref, s2a_ref, fa8a_ref
    m = mt_ref[...]
    fa8_ref[...] = m.astype(jnp.float8_e4m3fn)
    mb = m.astype(jnp.bfloat16)
    u1 = jnp.dot(mb, s1_ref[...], preferred_element_type=jnp.float32)
    u1 = u1 + b1_ref[...]
    u1_ref[...] = u1
    s2_ref[...] = jnp.dot(u1.astype(jnp.bfloat16), w2_ref[...],
                          preferred_element_type=jnp.float32).astype(
                              jnp.float8_e4m3fn)


def _pass2_body(wa_ref, fa8_ref, s2_ref, u1_ref, b2_ref, o_ref):
    wa = wa_ref[0, 0]
    u2 = jnp.dot(fa8_ref[...], s2_ref[...],
                 preferred_element_type=jnp.float32)
    u2 = u2 + b2_ref[...]
    o_ref[...] = (u1_ref[...] + u2 * wa) * 0.5


def kernel(feature, A, W1, b1, W2, b2, weight_b, weight_a):
    f_bf = feature.astype(jnp.bfloat16)
    w1_bf = W1.astype(jnp.bfloat16)
    w2_bf = W2.astype(jnp.bfloat16)
    b1_2d = b1.reshape(1, F)
    b2_2d = b2.reshape(1, F)

    m_tail = _sc_merge(A, weight_b)

    u1, s2, fa8, s1 = pl.pallas_call(
        _head_body,
        grid=(NBH,),
        in_specs=[
            pl.BlockSpec(memory_space=pltpu.SMEM),
            pl.BlockSpec((1, TM, N), lambda i: (0, i, 0)),
            pl.BlockSpec((1, TM, N), lambda i: (1, i, 0)),
            pl.BlockSpec((N, F), lambda i: (0, 0)),
            pl.BlockSpec((F, F), lambda i: (0, 0)),
            pl.BlockSpec((1, F), lambda i: (0, 0)),
            pl.BlockSpec((F, F), lambda i: (0, 0)),
        ],
        out_specs=[
            pl.BlockSpec((TM, F), lambda i: (i, 0)),
            pl.BlockSpec((TM, F), lambda i: (i, 0)),
            pl.BlockSpec((TM, N), lambda i: (i, 0)),
            pl.BlockSpec((N, F), lambda i: (0, 0)),
        ],
        out_shape=[
            jax.ShapeDtypeStruct((N, F), jnp.float32),
            jax.ShapeDtypeStruct((N, F), jnp.float8_e4m3fn),
            jax.ShapeDtypeStruct((N, N), jnp.float8_e4m3fn),
            jax.ShapeDtypeStruct((N, F), jnp.bfloat16),
        ],
        scratch_shapes=[pltpu.VMEM((N, F), jnp.bfloat16)],
    )(weight_b, A, A, f_bf, w1_bf, b1_2d, w2_bf)

    u1, s2, fa8 = pl.pallas_call(
        _tail_body,
        grid=(NBT,),
        in_specs=[
            pl.BlockSpec((TM, N), lambda i: (i, 0)),
            pl.BlockSpec((N, F), lambda i: (0, 0)),
            pl.BlockSpec((1, F), lambda i: (0, 0)),
            pl.BlockSpec((F, F), lambda i: (0, 0)),
            pl.BlockSpec(memory_space=pltpu.MemorySpace.HBM),
            pl.BlockSpec(memory_space=pltpu.MemorySpace.HBM),
            pl.BlockSpec(memory_space=pltpu.MemorySpace.HBM),
        ],
        out_specs=[
            pl.BlockSpec((TM, F), lambda i: (i + NBH, 0)),
            pl.BlockSpec((TM, F), lambda i: (i + NBH, 0)),
            pl.BlockSpec((TM, N), lambda i: (i + NBH, 0)),
        ],
        out_shape=[
            jax.ShapeDtypeStruct((N, F), jnp.float32),
            jax.ShapeDtypeStruct((N, F), jnp.float8_e4m3fn),
            jax.ShapeDtypeStruct((N, N), jnp.float8_e4m3fn),
        ],
        input_output_aliases={4: 0, 5: 1, 6: 2},
    )(m_tail, s1, b1_2d, w2_bf, u1, s2, fa8)

    out = pl.pallas_call(
        _pass2_body,
        grid=(NB,),
        in_specs=[
            pl.BlockSpec(memory_space=pltpu.SMEM),
            pl.BlockSpec((TM, N), lambda i: (i, 0)),
            pl.BlockSpec((N, F), lambda i: (0, 0)),
            pl.BlockSpec((TM, F), lambda i: (i, 0)),
            pl.BlockSpec((1, F), lambda i: (0, 0)),
        ],
        out_specs=pl.BlockSpec((TM, F), lambda i: (i, 0)),
        out_shape=jax.ShapeDtypeStruct((N, F), jnp.float32),
    )(weight_a, fa8, s2, u1, b2_2d)

    return out


# speedup vs baseline: 1.0009x; 1.0009x over previous
import functools

import jax
import jax.numpy as jnp
from jax import lax
from jax.experimental import pallas as pl
from jax.experimental.pallas import tpu as pltpu
from jax.experimental.pallas import tpu_sc as plsc

N = 10000
F = 128
TM = 200
TAIL = 1600
HEAD = N - TAIL
NBH = HEAD // TM
NBT = TAIL // TM
NB = N // TM

NWORK = 32
RPW = TAIL // NWORK
NL = 16
RG = 2


def _sc_merge_body(a_ref, wb_ref, out_ref,
                   b0_ref, b1_ref, m_ref, wbv_ref,
                   sem_a, sem_b, sem_o):
    wid = lax.axis_index("s") * 2 + lax.axis_index("c")
    pltpu.sync_copy(wb_ref, wbv_ref)
    v0 = wbv_ref[0, :]
    v1 = wbv_ref[1, :]
    base = wid * RPW

    def in_a(k, slot):
        return pltpu.make_async_copy(
            a_ref.at[0, pl.ds(HEAD + base + k * RG, RG), :],
            b0_ref.at[slot], sem_a.at[slot])

    def in_b(k, slot):
        return pltpu.make_async_copy(
            a_ref.at[1, pl.ds(HEAD + base + k * RG, RG), :],
            b1_ref.at[slot], sem_b.at[slot])

    def out_c(k, slot):
        return pltpu.make_async_copy(
            m_ref.at[slot], out_ref.at[pl.ds(base + k * RG, RG), :],
            sem_o.at[slot])

    in_a(0, 0).start()
    in_b(0, 0).start()

    def row_step(k, _):
        slot = lax.rem(k, 2)
        nslot = lax.rem(k + 1, 2)

        @pl.when(k >= 2)
        def _():
            out_c(k - 2, slot).wait()

        in_a(k, slot).wait()
        in_b(k, slot).wait()

        @pl.when(k < RPW // RG - 1)
        def _():
            in_a(k + 1, nslot).start()
            in_b(k + 1, nslot).start()

        def col_step(j, _):
            for r in range(RG):
                for c in range(25):
                    sl = pl.ds((j * 25 + c) * NL, NL)
                    m_ref[slot, r, sl] = (b0_ref[slot, r, sl] * v0
                                          + b1_ref[slot, r, sl] * v1)
            return ()

        lax.fori_loop(0, N // NL // 25, col_step, ())
        out_c(k, slot).start()
        return ()

    lax.fori_loop(0, RPW // RG, row_step, ())
    out_c(RPW // RG - 2, 0).wait()
    out_c(RPW // RG - 1, 1).wait()


def _sc_merge(A, weight_b):
    wbv = jnp.broadcast_to(weight_b.reshape(2, 1), (2, NL)).astype(jnp.float32)
    mesh = plsc.VectorSubcoreMesh(core_axis_name="c", subcore_axis_name="s")
    k = functools.partial(
        pl.kernel,
        out_type=jax.ShapeDtypeStruct((TAIL, N), jnp.float32),
        mesh=mesh,
        scratch_types=[
            pltpu.VMEM((2, RG, N), jnp.float32),
            pltpu.VMEM((2, RG, N), jnp.float32),
            pltpu.VMEM((2, RG, N), jnp.float32),
            pltpu.VMEM((2, NL), jnp.float32),
            pltpu.SemaphoreType.DMA((2,)),
            pltpu.SemaphoreType.DMA((2,)),
            pltpu.SemaphoreType.DMA((2,)),
        ],
    )(_sc_merge_body)
    return k(A, wbv)


def _head_body(wb_ref, a0_ref, a1_ref, f_ref, w1_ref, b1_ref, w2_ref,
               u1_ref, s2_ref, fa8_ref, s1out_ref, s1_ref):
    @pl.when(pl.program_id(0) == 0)
    def _():
        s1 = jnp.dot(f_ref[...], w1_ref[...],
                     preferred_element_type=jnp.float32).astype(jnp.bfloat16)
        s1_ref[...] = s1
        s1out_ref[...] = s1

    wb0 = wb_ref[0, 0]
    wb1 = wb_ref[1, 0]
    m = a0_ref[0] * wb0 + a1_ref[0] * wb1
    fa8_ref[...] = m.astype(jnp.float8_e4m3fn)
    mb = m.astype(jnp.bfloat16)
    u1 = jnp.dot(mb, s1_ref[...], preferred_element_type=jnp.float32)
    u1 = u1 + b1_ref[...]
    u1_ref[...] = u1
    s2_ref[...] = jnp.dot(u1.astype(jnp.bfloat16), w2_ref[...],
                          preferred_element_type=jnp.float32).astype(
                              jnp.float8_e4m3fn)


def _tail_body(mt_ref, s1_ref, b1_ref, w2_ref, u1a_ref, s2a_ref, fa8a_ref,
               u1_ref, s2_ref, fa8_ref):
    del u1a_ref, s2a_ref, fa8a_ref
    m = mt_ref[...]
    fa8_ref[...] = m.astype(jnp.float8_e4m3fn)
    mb = m.astype(jnp.bfloat16)
    u1 = jnp.dot(mb, s1_ref[...], preferred_element_type=jnp.float32)
    u1 = u1 + b1_ref[...]
    u1_ref[...] = u1
    s2_ref[...] = jnp.dot(u1.astype(jnp.bfloat16), w2_ref[...],
                          preferred_element_type=jnp.float32).astype(
                              jnp.float8_e4m3fn)


def _pass2_body(wa_ref, fa8_ref, s2_ref, u1_ref, b2_ref, o_ref):
    wa = wa_ref[0, 0]
    u2 = jnp.dot(fa8_ref[...], s2_ref[...],
                 preferred_element_type=jnp.float32)
    u2 = u2 + b2_ref[...]
    o_ref[...] = (u1_ref[...] + u2 * wa) * 0.5


def kernel(feature, A, W1, b1, W2, b2, weight_b, weight_a):
    f_bf = feature.astype(jnp.bfloat16)
    w1_bf = W1.astype(jnp.bfloat16)
    w2_bf = W2.astype(jnp.bfloat16)
    b1_2d = b1.reshape(1, F)
    b2_2d = b2.reshape(1, F)

    m_tail = _sc_merge(A, weight_b)

    u1, s2, fa8, s1 = pl.pallas_call(
        _head_body,
        grid=(NBH,),
        in_specs=[
            pl.BlockSpec(memory_space=pltpu.SMEM),
            pl.BlockSpec((1, TM, N), lambda i: (0, i, 0)),
            pl.BlockSpec((1, TM, N), lambda i: (1, i, 0)),
            pl.BlockSpec((N, F), lambda i: (0, 0)),
            pl.BlockSpec((F, F), lambda i: (0, 0)),
            pl.BlockSpec((1, F), lambda i: (0, 0)),
            pl.BlockSpec((F, F), lambda i: (0, 0)),
        ],
        out_specs=[
            pl.BlockSpec((TM, F), lambda i: (i, 0)),
            pl.BlockSpec((TM, F), lambda i: (i, 0)),
            pl.BlockSpec((TM, N), lambda i: (i, 0)),
            pl.BlockSpec((N, F), lambda i: (0, 0)),
        ],
        out_shape=[
            jax.ShapeDtypeStruct((N, F), jnp.float32),
            jax.ShapeDtypeStruct((N, F), jnp.float8_e4m3fn),
            jax.ShapeDtypeStruct((N, N), jnp.float8_e4m3fn),
            jax.ShapeDtypeStruct((N, F), jnp.bfloat16),
        ],
        scratch_shapes=[pltpu.VMEM((N, F), jnp.bfloat16)],
    )(weight_b, A, A, f_bf, w1_bf, b1_2d, w2_bf)

    u1, s2, fa8 = pl.pallas_call(
        _tail_body,
        grid=(NBT,),
        in_specs=[
            pl.BlockSpec((TM, N), lambda i: (i, 0)),
            pl.BlockSpec((N, F), lambda i: (0, 0)),
            pl.BlockSpec((1, F), lambda i: (0, 0)),
            pl.BlockSpec((F, F), lambda i: (0, 0)),
            pl.BlockSpec(memory_space=pltpu.MemorySpace.HBM),
            pl.BlockSpec(memory_space=pltpu.MemorySpace.HBM),
            pl.BlockSpec(memory_space=pltpu.MemorySpace.HBM),
        ],
        out_specs=[
            pl.BlockSpec((TM, F), lambda i: (i + NBH, 0)),
            pl.BlockSpec((TM, F), lambda i: (i + NBH, 0)),
            pl.BlockSpec((TM, N), lambda i: (i + NBH, 0)),
        ],
        out_shape=[
            jax.ShapeDtypeStruct((N, F), jnp.float32),
            jax.ShapeDtypeStruct((N, F), jnp.float8_e4m3fn),
            jax.ShapeDtypeStruct((N, N), jnp.float8_e4m3fn),
        ],
        input_output_aliases={4: 0, 5: 1, 6: 2},
    )(m_tail, s1, b1_2d, w2_bf, u1, s2, fa8)

    out = pl.pallas_call(
        _pass2_body,
        grid=(NB,),
        in_specs=[
            pl.BlockSpec(memory_space=pltpu.SMEM),
            pl.BlockSpec((TM, N), lambda i: (i, 0)),
            pl.BlockSpec((N, F), lambda i: (0, 0)),
            pl.BlockSpec((TM, F), lambda i: (i, 0)),
            pl.BlockSpec((1, F), lambda i: (0, 0)),
        ],
        out_specs=pl.BlockSpec((TM, F), lambda i: (i, 0)),
        out_shape=jax.ShapeDtypeStruct((N, F), jnp.float32),
    )(weight_a, fa8, s2, u1, b2_2d)

    return out


# speedup vs baseline: 1.1914x; 1.1903x over previous
import jax
import jax.numpy as jnp
from jax.experimental import pallas as pl
from jax.experimental.pallas import tpu as pltpu

N = 10000
F = 128
TM = 200
NB = N // TM


def _body(wb_ref, wa_ref, a0_ref, a1_ref, f_ref, w1_ref, b1_ref, w2_ref,
          b2_ref, fa8_ref, o_ref,
          s1_ref, u1_ref, s2_ref, spill_ref, fetch_ref, sem_out, sem_in):
    i = pl.program_id(0)

    @pl.when(i == 0)
    def _():
        s1_ref[...] = jnp.dot(f_ref[...], w1_ref[...],
                              preferred_element_type=jnp.float32
                              ).astype(jnp.bfloat16)

    @pl.when(i < NB)
    def _phase1():
        slot = jax.lax.rem(i, 2)

        @pl.when(i >= 2)
        def _():
            pltpu.make_async_copy(
                spill_ref.at[slot],
                fa8_ref.at[pl.ds((i - 2) * TM, TM), :],
                sem_out.at[slot]).wait()

        wb0 = wb_ref[0, 0]
        wb1 = wb_ref[1, 0]
        m = a0_ref[0] * wb0 + a1_ref[0] * wb1
        spill_ref[slot] = m.astype(jnp.float8_e4m3fn)
        pltpu.make_async_copy(
            spill_ref.at[slot],
            fa8_ref.at[pl.ds(i * TM, TM), :],
            sem_out.at[slot]).start()
        mb = m.astype(jnp.bfloat16)
        u1 = jnp.dot(mb, s1_ref[...], preferred_element_type=jnp.float32)
        u1 = u1 + b1_ref[...]
        u1_ref[pl.ds(i * TM, TM), :] = u1
        s2_ref[pl.ds(i * TM, TM), :] = jnp.dot(
            u1.astype(jnp.bfloat16), w2_ref[...],
            preferred_element_type=jnp.float32).astype(jnp.float8_e4m3fn)

    @pl.when(i >= NB)
    def _phase2():
        j = i - NB
        slot = jax.lax.rem(j, 2)

        @pl.when(j < 2)
        def _():
            pltpu.make_async_copy(
                spill_ref.at[slot],
                fa8_ref.at[pl.ds((NB - 2 + j) * TM, TM), :],
                sem_out.at[slot]).wait()

        @pl.when(j == 0)
        def _():
            pltpu.make_async_copy(
                fa8_ref.at[pl.ds(0, TM), :],
                fetch_ref.at[0],
                sem_in.at[0]).start()

        @pl.when(j < NB - 1)
        def _():
            nxt = jax.lax.rem(j + 1, 2)
            pltpu.make_async_copy(
                fa8_ref.at[pl.ds((j + 1) * TM, TM), :],
                fetch_ref.at[nxt],
                sem_in.at[nxt]).start()

        pltpu.make_async_copy(
            fa8_ref.at[pl.ds(j * TM, TM), :],
            fetch_ref.at[slot],
            sem_in.at[slot]).wait()

        wa = wa_ref[0, 0]
        u2 = jnp.dot(fetch_ref[slot], s2_ref[...],
                     preferred_element_type=jnp.float32)
        u2 = u2 + b2_ref[...]
        o_ref[...] = (u1_ref[pl.ds(j * TM, TM), :] + u2 * wa) * 0.5


def kernel(feature, A, W1, b1, W2, b2, weight_b, weight_a):
    f_bf = feature.astype(jnp.bfloat16)
    w1_bf = W1.astype(jnp.bfloat16)
    w2_bf = W2.astype(jnp.bfloat16)
    b1_2d = b1.reshape(1, F)
    b2_2d = b2.reshape(1, F)

    _, out = pl.pallas_call(
        _body,
        grid=(2 * NB,),
        in_specs=[
            pl.BlockSpec(memory_space=pltpu.SMEM),
            pl.BlockSpec(memory_space=pltpu.SMEM),
            pl.BlockSpec((1, TM, N), lambda i: (0, jnp.minimum(i, NB - 1), 0)),
            pl.BlockSpec((1, TM, N), lambda i: (1, jnp.minimum(i, NB - 1), 0)),
            pl.BlockSpec((N, F), lambda i: (0, 0)),
            pl.BlockSpec((F, F), lambda i: (0, 0)),
            pl.BlockSpec((1, F), lambda i: (0, 0)),
            pl.BlockSpec((F, F), lambda i: (0, 0)),
            pl.BlockSpec((1, F), lambda i: (0, 0)),
        ],
        out_specs=[
            pl.BlockSpec(memory_space=pltpu.MemorySpace.HBM),
            pl.BlockSpec((TM, F), lambda i: (jnp.maximum(i - NB, 0), 0)),
        ],
        out_shape=[
            jax.ShapeDtypeStruct((N, N), jnp.float8_e4m3fn),
            jax.ShapeDtypeStruct((N, F), jnp.float32),
        ],
        scratch_shapes=[
            pltpu.VMEM((N, F), jnp.bfloat16),
            pltpu.VMEM((N, F), jnp.float32),
            pltpu.VMEM((N, F), jnp.float8_e4m3fn),
            pltpu.VMEM((2, TM, N), jnp.float8_e4m3fn),
            pltpu.VMEM((2, TM, N), jnp.float8_e4m3fn),
            pltpu.SemaphoreType.DMA((2,)),
            pltpu.SemaphoreType.DMA((2,)),
        ],
    )(weight_b, weight_a, A, A, f_bf, w1_bf, b1_2d, w2_bf, b2_2d)

    return out
